# Initial kernel scaffold; baseline (speedup 1.0000x reference)
import jax
import jax.numpy as jnp
from jax.experimental import pallas as pl


def kernel(x, token_emb, pos_emb):
    raise NotImplementedError("write your pallas kernel here")


# speedup vs baseline: 4.8575x; 4.8575x over previous
import functools

import jax
import jax.numpy as jnp
from jax import lax
from jax.experimental import pallas as pl
from jax.experimental.pallas import tpu as pltpu
from jax.experimental.pallas import tpu_sc as plsc

MAXLEN = 200
EMBED_DIM = 32
BATCH = 4096

_NC = 2
_NS = 16
_NW = _NC * _NS
_SEQ_PER_W = BATCH // _NW
_CSEQ = 8
_NCHUNK = _SEQ_PER_W // _CSEQ
_CROWS = _CSEQ * MAXLEN
_GB = 128

_BURSTS = [(j * _GB, _GB) for j in range(_CROWS // _GB)]
if _CROWS % _GB:
    _BURSTS.append((_CROWS - _CROWS % _GB, _CROWS % _GB))


def _make_kernel():
    mesh = plsc.VectorSubcoreMesh(core_axis_name="c", subcore_axis_name="s")

    @functools.partial(
        pl.kernel,
        mesh=mesh,
        compiler_params=pltpu.CompilerParams(use_tc_tiling_on_sc=False),
        out_type=jax.ShapeDtypeStruct((BATCH * MAXLEN, EMBED_DIM), jnp.float32),
        scratch_types=[
            pltpu.VMEM((_CROWS,), jnp.int32),
            pltpu.VMEM((_CROWS, EMBED_DIM), jnp.float32),
            pltpu.VMEM((MAXLEN, EMBED_DIM), jnp.float32),
            pltpu.SemaphoreType.DMA,
        ],
    )
    def k(idx_hbm, tok_hbm, pos_hbm, out_hbm, idx_v, rows_v, pos_v, sem):
        wid = lax.axis_index("s") * _NC + lax.axis_index("c")
        base = wid * (_SEQ_PER_W * MAXLEN)
        pltpu.sync_copy(pos_hbm, pos_v)

        def chunk_body(g, carry):
            row0 = base + g * _CROWS
            pltpu.sync_copy(idx_hbm.at[pl.ds(row0, _CROWS)], idx_v)
            copies = []
            for off, ln in _BURSTS:
                copies.append(pltpu.async_copy(
                    tok_hbm.at[idx_v.at[pl.ds(off, ln)]],
                    rows_v.at[pl.ds(off, ln), :],
                    sem))
            for c in copies:
                c.wait()

            def pos_body(p, c2):
                pv0 = pos_v[p, pl.ds(0, 16)]
                pv1 = pos_v[p, pl.ds(16, 16)]
                for s2 in range(_CSEQ):
                    r = s2 * MAXLEN + p
                    rows_v[r, pl.ds(0, 16)] = rows_v[r, pl.ds(0, 16)] + pv0
                    rows_v[r, pl.ds(16, 16)] = rows_v[r, pl.ds(16, 16)] + pv1
                return c2

            lax.fori_loop(0, MAXLEN, pos_body, 0)
            pltpu.sync_copy(rows_v, out_hbm.at[pl.ds(row0, _CROWS)])
            return carry

        lax.fori_loop(0, _NCHUNK, chunk_body, 0)

    return k


_sc_kernel = _make_kernel()


def kernel(x, token_emb, pos_emb):
    idx = x.reshape(-1).astype(jnp.int32)
    out = _sc_kernel(idx, token_emb, pos_emb)
    return out.reshape(BATCH, MAXLEN, EMBED_DIM)


# speedup vs baseline: 5.1784x; 1.0661x over previous
import functools

import jax
import jax.numpy as jnp
from jax import lax
from jax.experimental import pallas as pl
from jax.experimental.pallas import tpu as pltpu
from jax.experimental.pallas import tpu_sc as plsc

MAXLEN = 200
EMBED_DIM = 32
BATCH = 4096

_NC = 2
_NS = 16
_NW = _NC * _NS
_SEQ_PER_W = BATCH // _NW
_CSEQ = 8
_NCHUNK = _SEQ_PER_W // _CSEQ
_CROWS = _CSEQ * MAXLEN
_GB = 128

_BURSTS = [(j * _GB, _GB) for j in range(_CROWS // _GB)]
if _CROWS % _GB:
    _BURSTS.append((_CROWS - _CROWS % _GB, _CROWS % _GB))


def _make_kernel():
    mesh = plsc.VectorSubcoreMesh(core_axis_name="c", subcore_axis_name="s")

    @functools.partial(
        pl.kernel,
        mesh=mesh,
        compiler_params=pltpu.CompilerParams(use_tc_tiling_on_sc=False),
        out_type=jax.ShapeDtypeStruct((BATCH * MAXLEN, EMBED_DIM), jnp.float32),
        scratch_types=[
            pltpu.VMEM((2, _CROWS), jnp.int32),
            pltpu.VMEM((2, _CROWS, EMBED_DIM), jnp.float32),
            pltpu.VMEM((MAXLEN, EMBED_DIM), jnp.float32),
            pltpu.SemaphoreType.DMA,
            pltpu.SemaphoreType.DMA,
            pltpu.SemaphoreType.DMA,
            pltpu.SemaphoreType.DMA,
        ],
    )
    def k(idx_hbm, tok_hbm, pos_hbm, out_hbm, idx_v, rows_v, pos_v,
          gsem0, gsem1, osem0, osem1):
        wid = lax.axis_index("s") * _NC + lax.axis_index("c")
        base = wid * (_SEQ_PER_W * MAXLEN)
        gsems = (gsem0, gsem1)
        osems = (osem0, osem1)
        pltpu.sync_copy(pos_hbm, pos_v)

        def fire_gather(g):
            p = g % 2
            row0 = base + g * _CROWS
            pltpu.sync_copy(idx_hbm.at[pl.ds(row0, _CROWS)], idx_v.at[p])
            copies = []
            for off, ln in _BURSTS:
                copies.append(pltpu.async_copy(
                    tok_hbm.at[idx_v.at[p].at[pl.ds(off, ln)]],
                    rows_v.at[p].at[pl.ds(off, ln), :],
                    gsems[p]))
            return copies

        def add_pos(g):
            p = g % 2
            buf = rows_v.at[p]

            def pos_body(pos, c2):
                pv0 = pos_v[pos, pl.ds(0, 16)]
                pv1 = pos_v[pos, pl.ds(16, 16)]
                for s2 in range(_CSEQ):
                    r = s2 * MAXLEN + pos
                    buf[r, pl.ds(0, 16)] = buf[r, pl.ds(0, 16)] + pv0
                    buf[r, pl.ds(16, 16)] = buf[r, pl.ds(16, 16)] + pv1
                return c2

            lax.fori_loop(0, MAXLEN, pos_body, 0)

        gath = {0: fire_gather(0)}
        outc = {}
        for g in range(_NCHUNK):
            if g + 1 < _NCHUNK:
                if g - 1 >= 0:
                    outc.pop(g - 1).wait()
                gath[g + 1] = fire_gather(g + 1)
            for c in gath.pop(g):
                c.wait()
            add_pos(g)
            p = g % 2
            outc[g] = pltpu.async_copy(
                rows_v.at[p], out_hbm.at[pl.ds(base + g * _CROWS, _CROWS)],
                osems[p])
        for g in sorted(outc):
            outc.pop(g).wait()

    return k


_sc_kernel = _make_kernel()


def kernel(x, token_emb, pos_emb):
    idx = x.reshape(-1).astype(jnp.int32)
    out = _sc_kernel(idx, token_emb, pos_emb)
    return out.reshape(BATCH, MAXLEN, EMBED_DIM)


# speedup vs baseline: 7.3056x; 1.4108x over previous
import functools

import jax
import jax.numpy as jnp
from jax import lax
from jax.experimental import pallas as pl
from jax.experimental.pallas import tpu as pltpu
from jax.experimental.pallas import tpu_sc as plsc

MAXLEN = 200
VOCAB = 100000
EMBED_DIM = 32
BATCH = 4096

_NC = 2
_NS = 16
_NW = _NC * _NS
_LANES = 16
_BTILE = 128
_NBT = BATCH // _BTILE
_DT = EMBED_DIM // 8
_VPB = _BTILE // _LANES

def _make_kernel():
    mesh = plsc.VectorSubcoreMesh(core_axis_name="c", subcore_axis_name="s")

    @functools.partial(
        pl.kernel,
        mesh=mesh,
        compiler_params=pltpu.CompilerParams(
            use_tc_tiling_on_sc=False, needs_layout_passes=False),
        out_type=jax.ShapeDtypeStruct((MAXLEN, _DT, _NBT, 8, _BTILE),
                                      jnp.float32),
        scratch_types=[
            pltpu.VMEM((VOCAB,), jnp.float32),
            pltpu.VMEM((BATCH,), jnp.int32),
            pltpu.VMEM((2, _NBT, _BTILE), jnp.float32),
            pltpu.VMEM((MAXLEN,), jnp.float32),
            pltpu.VMEM_SHARED((MAXLEN // 4, BATCH), jnp.int32),
            pltpu.SemaphoreType.DMA,
            pltpu.SemaphoreType.DMA,
        ],
    )
    def k(xT_hbm, tokT_hbm, posT_hbm, out_hbm,
          trow_v, xcol_v, slab_v, pos_v, xsh, sem0, sem1):
        wid = lax.axis_index("s") * _NC + lax.axis_index("c")
        tr = wid // 8
        fr = wid % 8
        sems = (sem0, sem1)

        pltpu.sync_copy(tokT_hbm.at[wid], trow_v)
        pltpu.sync_copy(posT_hbm.at[wid], pos_v)

        sid = lax.axis_index("s")
        half = MAXLEN // 4

        def do_seq(s, srow, par):
            pltpu.sync_copy(xsh.at[srow], xcol_v)
            pv = plsc.load_gather(pos_v, [jnp.full((_LANES,), s, jnp.int32)])
            buf = slab_v.at[par]

            def gather_body(j, c):
                for u in range(_VPB):
                    idx16 = xcol_v[pl.ds(j * _BTILE + u * _LANES, _LANES)]
                    vals = plsc.load_gather(trow_v, [idx16])
                    buf[j, pl.ds(u * _LANES, _LANES)] = vals + pv
                return c

            lax.fori_loop(0, _NBT, gather_body, 0)
            pltpu.async_copy(buf, out_hbm.at[s, tr, :, fr, :], sems[par])

        for phase in range(4):
            @pl.when(sid < 2)
            def _():
                st = sid * 4
                pltpu.sync_copy(xT_hbm.at[pl.ds(phase * half + st, 4)],
                                xsh.at[pl.ds(st, 4)])

            @pl.when(sid >= 2)
            def _():
                st = 8 + (sid - 2) * 3
                pltpu.sync_copy(xT_hbm.at[pl.ds(phase * half + st, 3)],
                                xsh.at[pl.ds(st, 3)])

            plsc.subcore_barrier()

            def seq_pair(kk, c):
                for par in range(2):
                    @pl.when((kk > 0) | (phase > 0))
                    def _():
                        pltpu.make_async_copy(
                            slab_v.at[par], out_hbm.at[0, tr, :, fr, :],
                            sems[par]).wait()

                    do_seq(phase * half + kk * 2 + par, kk * 2 + par, par)
                return c

            lax.fori_loop(0, half // 2, seq_pair, 0)
            plsc.subcore_barrier()

        for par in range(2):
            pltpu.make_async_copy(
                slab_v.at[par], out_hbm.at[0, tr, :, fr, :],
                sems[par]).wait()

    return k


_sc_kernel = _make_kernel()


def kernel(x, token_emb, pos_emb):
    xT = x.T.astype(jnp.int32)
    tokT = token_emb.T
    posT = pos_emb.T
    out5 = _sc_kernel(xT, tokT, posT)
    return (out5.transpose(2, 4, 0, 1, 3)
            .reshape(BATCH, MAXLEN, EMBED_DIM))


# speedup vs baseline: 15.5489x; 2.1283x over previous
import functools

import jax
import jax.numpy as jnp
from jax import lax
from jax.experimental import pallas as pl
from jax.experimental.pallas import tpu as pltpu
from jax.experimental.pallas import tpu_sc as plsc

MAXLEN = 200
VOCAB = 100000
EMBED_DIM = 32
BATCH = 4096

_NC = 2
_NS = 16
_NW = _NC * _NS
_LANES = 16
_BTILE = 128
_NBT = BATCH // _BTILE
_DT = EMBED_DIM // 8
_VPB = _BTILE // _LANES

def _make_kernel():
    mesh = plsc.VectorSubcoreMesh(core_axis_name="c", subcore_axis_name="s")

    @functools.partial(
        pl.kernel,
        mesh=mesh,
        compiler_params=pltpu.CompilerParams(
            use_tc_tiling_on_sc=False, needs_layout_passes=False),
        out_type=jax.ShapeDtypeStruct((MAXLEN, _DT, _NBT, 8, _BTILE),
                                      jnp.float32),
        scratch_types=[
            pltpu.VMEM((VOCAB,), jnp.float32),
            pltpu.VMEM((BATCH,), jnp.int32),
            pltpu.VMEM((2, _NBT, _BTILE), jnp.float32),
            pltpu.VMEM((MAXLEN,), jnp.float32),
            pltpu.VMEM_SHARED((MAXLEN // 4, BATCH), jnp.int32),
            pltpu.SemaphoreType.DMA,
            pltpu.SemaphoreType.DMA,
        ],
    )
    def k(xT_hbm, tokT_hbm, posT_hbm, out_hbm,
          trow_v, xcol_v, slab_v, pos_v, xsh, sem0, sem1):
        wid = lax.axis_index("s") * _NC + lax.axis_index("c")
        tr = wid // 8
        fr = wid % 8
        sems = (sem0, sem1)

        pltpu.sync_copy(tokT_hbm.at[wid], trow_v)
        pltpu.sync_copy(posT_hbm.at[wid], pos_v)

        sid = lax.axis_index("s")
        half = MAXLEN // 4

        def do_seq(s, srow, par):
            pltpu.sync_copy(xsh.at[srow], xcol_v)
            pv = plsc.load_gather(pos_v, [jnp.full((_LANES,), s, jnp.int32)])
            buf = slab_v.at[par]

            @plsc.parallel_loop(0, _NBT, 1, unroll=2)
            def gather_body(j):
                for u in range(_VPB):
                    idx16 = xcol_v[pl.ds(j * _BTILE + u * _LANES, _LANES)]
                    vals = plsc.load_gather(trow_v, [idx16])
                    buf[j, pl.ds(u * _LANES, _LANES)] = vals + pv
            pltpu.async_copy(buf, out_hbm.at[s, tr, :, fr, :], sems[par])

        for phase in range(4):
            @pl.when(sid < 2)
            def _():
                st = sid * 4
                pltpu.sync_copy(xT_hbm.at[pl.ds(phase * half + st, 4)],
                                xsh.at[pl.ds(st, 4)])

            @pl.when(sid >= 2)
            def _():
                st = 8 + (sid - 2) * 3
                pltpu.sync_copy(xT_hbm.at[pl.ds(phase * half + st, 3)],
                                xsh.at[pl.ds(st, 3)])

            plsc.subcore_barrier()

            def seq_pair(kk, c):
                for par in range(2):
                    @pl.when((kk > 0) | (phase > 0))
                    def _():
                        pltpu.make_async_copy(
                            slab_v.at[par], out_hbm.at[0, tr, :, fr, :],
                            sems[par]).wait()

                    do_seq(phase * half + kk * 2 + par, kk * 2 + par, par)
                return c

            lax.fori_loop(0, half // 2, seq_pair, 0)
            plsc.subcore_barrier()

        for par in range(2):
            pltpu.make_async_copy(
                slab_v.at[par], out_hbm.at[0, tr, :, fr, :],
                sems[par]).wait()

    return k


_sc_kernel = _make_kernel()


def kernel(x, token_emb, pos_emb):
    xT = x.T.astype(jnp.int32)
    tokT = token_emb.T
    posT = pos_emb.T
    out5 = _sc_kernel(xT, tokT, posT)
    return (out5.transpose(2, 4, 0, 1, 3)
            .reshape(BATCH, MAXLEN, EMBED_DIM))


# speedup vs baseline: 15.7758x; 1.0146x over previous
import functools

import jax
import jax.numpy as jnp
from jax import lax
from jax.experimental import pallas as pl
from jax.experimental.pallas import tpu as pltpu
from jax.experimental.pallas import tpu_sc as plsc

MAXLEN = 200
VOCAB = 100000
EMBED_DIM = 32
BATCH = 4096

_NC = 2
_NS = 16
_NW = _NC * _NS
_LANES = 16
_BTILE = 128
_NBT = BATCH // _BTILE
_DT = EMBED_DIM // 8
_VPB = _BTILE // _LANES

def _make_kernel():
    mesh = plsc.VectorSubcoreMesh(core_axis_name="c", subcore_axis_name="s")

    @functools.partial(
        pl.kernel,
        mesh=mesh,
        compiler_params=pltpu.CompilerParams(
            use_tc_tiling_on_sc=False, needs_layout_passes=False),
        out_type=jax.ShapeDtypeStruct((MAXLEN, _DT, _NBT, 8, _BTILE),
                                      jnp.float32),
        scratch_types=[
            pltpu.VMEM((VOCAB,), jnp.float32),
            pltpu.VMEM((BATCH,), jnp.int32),
            pltpu.VMEM((2, _NBT, _BTILE), jnp.float32),
            pltpu.VMEM((MAXLEN,), jnp.float32),
            pltpu.VMEM_SHARED((MAXLEN // 4, BATCH), jnp.int32),
            pltpu.SemaphoreType.DMA,
            pltpu.SemaphoreType.DMA,
        ],
    )
    def k(xT_hbm, tokT_hbm, posT_hbm, out_hbm,
          trow_v, xcol_v, slab_v, pos_v, xsh, sem0, sem1):
        wid = lax.axis_index("s") * _NC + lax.axis_index("c")
        tr = wid // 8
        fr = wid % 8
        sems = (sem0, sem1)

        pltpu.sync_copy(tokT_hbm.at[wid], trow_v)
        pltpu.sync_copy(posT_hbm.at[wid], pos_v)

        sid = lax.axis_index("s")
        half = MAXLEN // 4

        def do_seq(s, srow, par):
            pltpu.sync_copy(xsh.at[srow], xcol_v)
            pv = plsc.load_gather(pos_v, [jnp.full((_LANES,), s, jnp.int32)])
            buf = slab_v.at[par]

            @plsc.parallel_loop(0, _NBT, 1, unroll=4)
            def gather_body(j):
                for u in range(_VPB):
                    idx16 = xcol_v[pl.ds(j * _BTILE + u * _LANES, _LANES)]
                    vals = plsc.load_gather(trow_v, [idx16])
                    buf[j, pl.ds(u * _LANES, _LANES)] = vals + pv
            pltpu.async_copy(buf, out_hbm.at[s, tr, :, fr, :], sems[par])

        for phase in range(4):
            @pl.when(sid < 2)
            def _():
                st = sid * 4
                pltpu.sync_copy(xT_hbm.at[pl.ds(phase * half + st, 4)],
                                xsh.at[pl.ds(st, 4)])

            @pl.when(sid >= 2)
            def _():
                st = 8 + (sid - 2) * 3
                pltpu.sync_copy(xT_hbm.at[pl.ds(phase * half + st, 3)],
                                xsh.at[pl.ds(st, 3)])

            plsc.subcore_barrier()

            def seq_pair(kk, c):
                for par in range(2):
                    @pl.when((kk > 0) | (phase > 0))
                    def _():
                        pltpu.make_async_copy(
                            slab_v.at[par], out_hbm.at[0, tr, :, fr, :],
                            sems[par]).wait()

                    do_seq(phase * half + kk * 2 + par, kk * 2 + par, par)
                return c

            lax.fori_loop(0, half // 2, seq_pair, 0)
            plsc.subcore_barrier()

        for par in range(2):
            pltpu.make_async_copy(
                slab_v.at[par], out_hbm.at[0, tr, :, fr, :],
                sems[par]).wait()

    return k


_sc_kernel = _make_kernel()


def kernel(x, token_emb, pos_emb):
    xT = x.T.astype(jnp.int32)
    tokT = token_emb.T
    posT = pos_emb.T
    out5 = _sc_kernel(xT, tokT, posT)
    return (out5.transpose(2, 4, 0, 1, 3)
            .reshape(BATCH, MAXLEN, EMBED_DIM))
